# Initial kernel scaffold; baseline (speedup 1.0000x reference)
import jax
import jax.numpy as jnp
from jax.experimental import pallas as pl


def kernel(x, edge_index, Wn0, Ws0, b0, Wn1, Ws1, b1, Wn2, Ws2, b2):
    raise NotImplementedError("write your pallas kernel here")


# speedup vs baseline: 11.9265x; 11.9265x over previous
import functools

import jax
import jax.numpy as jnp
from jax import lax
from jax.experimental import pallas as pl
from jax.experimental.pallas import tpu as pltpu
from jax.experimental.pallas import tpu_sc as plsc

N = 10000
E = 320000
H = 128

NW = 32
EPW = E // NW
CH = 100
NCH = EPW // CH
SCH = 20
DBIG = 640
DLAST = N - 15 * DBIG
NPAD = 16 * DBIG


def _seg_sum(h, src3, dst3, with_deg: bool):
    mesh = plsc.VectorSubcoreMesh(core_axis_name="c", subcore_axis_name="s")
    out_type = [jax.ShapeDtypeStruct((2, N, H), jnp.float32)]
    if with_deg:
        out_type.append(jax.ShapeDtypeStruct((2, 1, NPAD), jnp.float32))
    scratch = [
        pltpu.VMEM_SHARED((N, H), jnp.float32),
        pltpu.VMEM((16, H), jnp.float32),
        pltpu.VMEM((SCH, CH), jnp.int32),
        pltpu.VMEM((SCH, CH), jnp.int32),
        pltpu.VMEM((CH, H), jnp.float32),
        pltpu.VMEM((CH, H), jnp.float32),
        pltpu.SemaphoreType.DMA,
        pltpu.SemaphoreType.DMA,
    ]
    if with_deg:
        scratch += [
            pltpu.VMEM_SHARED((NPAD,), jnp.float32),
            pltpu.VMEM((DBIG,), jnp.float32),
            pltpu.VMEM((112,), jnp.float32),
        ]

    def body(h_hbm, src_hbm, dst_hbm, *rest):
        if with_deg:
            (agg_out, deg_out, acc, zbuf, sidx, didx, rows0, rows1,
             semA, semB, accd, zdeg, ones) = rest
        else:
            (agg_out, acc, zbuf, sidx, didx, rows0, rows1,
             semA, semB) = rest
        c = lax.axis_index("c")
        s = lax.axis_index("s")
        w = c * 16 + s

        z16 = jnp.zeros((16,), jnp.float32)

        @pl.loop(0, 16)
        def _(i):
            for j in range(H // 16):
                zbuf[i, pl.ds(j * 16, 16)] = z16

        if with_deg:
            @pl.loop(0, DBIG // 16)
            def _(i):
                zdeg[pl.ds(i * 16, 16)] = z16
            o16 = jnp.ones((16,), jnp.float32)
            for i in range(112 // 16):
                ones[pl.ds(i * 16, 16)] = o16

        @pl.loop(0, DBIG // 16)
        def _(k):
            @pl.when(s * DBIG + k * 16 < N)
            def _():
                pltpu.sync_copy(zbuf, acc.at[pl.ds(s * DBIG + k * 16, 16)])
        if with_deg:
            pltpu.sync_copy(zdeg, accd.at[pl.ds(s * DBIG, DBIG)])
        plsc.subcore_barrier()

        def chunk(j, rows, sem):
            pltpu.make_async_copy(h_hbm.at[sidx.at[j]], rows, sem).wait()
            pltpu.sync_copy(rows, acc.at[didx.at[j]], add=True)
            if with_deg:
                pltpu.sync_copy(ones.at[pl.ds(0, CH)],
                                accd.at[didx.at[j]], add=True)

        for g in range(0, NCH, SCH):
            pltpu.sync_copy(src_hbm.at[w, pl.ds(g, SCH)], sidx)
            pltpu.sync_copy(dst_hbm.at[w, pl.ds(g, SCH)], didx)
            pltpu.async_copy(h_hbm.at[sidx.at[0]], rows0, semA)

            @pl.loop(0, SCH, step=2)
            def _(j):
                pltpu.async_copy(h_hbm.at[sidx.at[j + 1]], rows1, semB)
                chunk(j, rows0, semA)

                @pl.when(j + 2 < SCH)
                def _():
                    pltpu.async_copy(h_hbm.at[sidx.at[j + 2]], rows0, semA)
                chunk(j + 1, rows1, semB)

        plsc.subcore_barrier()

        @pl.when(s < 15)
        def _():
            pltpu.sync_copy(acc.at[pl.ds(s * DBIG, DBIG)],
                            agg_out.at[c, pl.ds(s * DBIG, DBIG)])

        @pl.when(s == 15)
        def _():
            pltpu.sync_copy(acc.at[pl.ds(15 * DBIG, DLAST)],
                            agg_out.at[c, pl.ds(15 * DBIG, DLAST)])
        if with_deg:
            pltpu.sync_copy(accd.at[pl.ds(s * DBIG, DBIG)],
                            deg_out.at[c, 0, pl.ds(s * DBIG, DBIG)])

    fn = pl.kernel(
        body, out_type=tuple(out_type), mesh=mesh, scratch_types=scratch,
        compiler_params=pltpu.CompilerParams(use_tc_tiling_on_sc=False))
    return fn(h, src3, dst3)


_RB = 2000


def _tc_layer_body(aA, aB, dA, dB, h_, wn, ws, bb, o, *, act):
    invd = 1.0 / jnp.maximum(dA[...] + dB[...], 1.0)
    agg = (aA[0] + aB[0]) * invd
    out = (jnp.dot(agg, wn[...], preferred_element_type=jnp.float32)
           + jnp.dot(h_[...], ws[...], preferred_element_type=jnp.float32)
           + bb[...])
    if act:
        out = jnp.maximum(out, 0.0)
    o[...] = out


def _tc_layer(aggp, degA, degB, h, Wn, Ws, b2d, act):
    row = lambda i: (i, 0)
    fixed = lambda i: (0, 0)
    return pl.pallas_call(
        functools.partial(_tc_layer_body, act=act),
        grid=(N // _RB,),
        in_specs=[
            pl.BlockSpec((1, _RB, H), lambda i: (0, i, 0)),
            pl.BlockSpec((1, _RB, H), lambda i: (1, i, 0)),
            pl.BlockSpec((_RB, 1), row),
            pl.BlockSpec((_RB, 1), row),
            pl.BlockSpec((_RB, H), row),
            pl.BlockSpec((H, H), fixed),
            pl.BlockSpec((H, H), fixed),
            pl.BlockSpec((1, H), fixed),
        ],
        out_specs=pl.BlockSpec((_RB, H), row),
        out_shape=jax.ShapeDtypeStruct((N, H), jnp.float32),
    )(aggp, aggp, degA, degB, h, Wn, Ws, b2d)


def _tc_final_body(aA, aB, dA, dB, h_, wn, ws, bb, o):
    invd = 1.0 / jnp.maximum(dA[...] + dB[...], 1.0)
    t = (aA[0] + aB[0]) * invd * wn[...] + h_[...] * ws[...]
    o[...] = jnp.sum(t, axis=1, keepdims=True) + bb[...]


def _tc_final(aggp, degA, degB, h, wn_row, ws_row, b11):
    row = lambda i: (i, 0)
    fixed = lambda i: (0, 0)
    return pl.pallas_call(
        _tc_final_body,
        grid=(N // _RB,),
        in_specs=[
            pl.BlockSpec((1, _RB, H), lambda i: (0, i, 0)),
            pl.BlockSpec((1, _RB, H), lambda i: (1, i, 0)),
            pl.BlockSpec((_RB, 1), row),
            pl.BlockSpec((_RB, 1), row),
            pl.BlockSpec((_RB, H), row),
            pl.BlockSpec((1, H), fixed),
            pl.BlockSpec((1, H), fixed),
            pl.BlockSpec((1, 1), fixed),
        ],
        out_specs=pl.BlockSpec((_RB, 1), row),
        out_shape=jax.ShapeDtypeStruct((N, 1), jnp.float32),
    )(aggp, aggp, degA, degB, h, wn_row, ws_row, b11)


def kernel(x, edge_index, Wn0, Ws0, b0, Wn1, Ws1, b1, Wn2, Ws2, b2):
    src3 = edge_index[0].reshape(NW, NCH, CH)
    dst3 = edge_index[1].reshape(NW, NCH, CH)

    agg0, degp = _seg_sum(x, src3, dst3, with_deg=True)
    degA = degp[0, 0, :N].reshape(N, 1)
    degB = degp[1, 0, :N].reshape(N, 1)

    h1 = _tc_layer(agg0, degA, degB, x,
                   Wn0, Ws0, b0.reshape(1, H), act=True)
    (agg1,) = _seg_sum(h1, src3, dst3, with_deg=False)
    h2 = _tc_layer(agg1, degA, degB, h1,
                   Wn1, Ws1, b1.reshape(1, H), act=True)
    (agg2,) = _seg_sum(h2, src3, dst3, with_deg=False)
    out = _tc_final(agg2, degA, degB, h2,
                    Wn2.reshape(1, H), Ws2.reshape(1, H), b2.reshape(1, 1))
    return out.reshape(N)


# speedup vs baseline: 14.7191x; 1.2341x over previous
import functools

import jax
import jax.numpy as jnp
from jax import lax
from jax.experimental import pallas as pl
from jax.experimental.pallas import tpu as pltpu
from jax.experimental.pallas import tpu_sc as plsc

N = 10000
E = 320000
H = 128

NW = 32
EPW = E // NW
CH = 125
NCH = EPW // CH
SCH = 16
CH1 = 80
NCH1 = EPW // CH1
SCH1 = 25
DBIG = 640
DLAST = N - 15 * DBIG
NPAD = 16 * DBIG

_SC_PARAMS = pltpu.CompilerParams(use_tc_tiling_on_sc=False)
_SC_PARAMS_GATHER = pltpu.CompilerParams(use_tc_tiling_on_sc=False,
                                         needs_layout_passes=False)


def _mesh():
    return plsc.VectorSubcoreMesh(core_axis_name="c", subcore_axis_name="s")


def _seg_sum(h, src3, dst3, with_deg: bool):
    out_type = [jax.ShapeDtypeStruct((2, N, H), jnp.float32)]
    if with_deg:
        out_type.append(jax.ShapeDtypeStruct((2, 1, NPAD), jnp.float32))
    scratch = [
        pltpu.VMEM_SHARED((N, H), jnp.float32),
        pltpu.VMEM((16, H), jnp.float32),
        pltpu.VMEM((SCH, CH), jnp.int32),
        pltpu.VMEM((SCH, CH), jnp.int32),
        pltpu.VMEM((CH, H), jnp.float32),
        pltpu.VMEM((CH, H), jnp.float32),
        pltpu.SemaphoreType.DMA,
        pltpu.SemaphoreType.DMA,
    ]
    if with_deg:
        scratch += [
            pltpu.VMEM_SHARED((NPAD,), jnp.float32),
            pltpu.VMEM((DBIG,), jnp.float32),
            pltpu.VMEM((128,), jnp.float32),
        ]

    def body(h_hbm, src_hbm, dst_hbm, *rest):
        if with_deg:
            (agg_out, deg_out, acc, zbuf, sidx, didx, rows0, rows1,
             semA, semB, accd, zdeg, ones) = rest
        else:
            (agg_out, acc, zbuf, sidx, didx, rows0, rows1,
             semA, semB) = rest
        c = lax.axis_index("c")
        s = lax.axis_index("s")
        w = c * 16 + s

        z16 = jnp.zeros((16,), jnp.float32)

        @pl.loop(0, 16)
        def _(i):
            for j in range(H // 16):
                zbuf[i, pl.ds(j * 16, 16)] = z16

        if with_deg:
            @pl.loop(0, DBIG // 16)
            def _(i):
                zdeg[pl.ds(i * 16, 16)] = z16
            o16 = jnp.ones((16,), jnp.float32)
            for i in range(128 // 16):
                ones[pl.ds(i * 16, 16)] = o16

        @pl.loop(0, DBIG // 16)
        def _(k):
            @pl.when(s * DBIG + k * 16 < N)
            def _():
                pltpu.sync_copy(zbuf, acc.at[pl.ds(s * DBIG + k * 16, 16)])
        if with_deg:
            pltpu.sync_copy(zdeg, accd.at[pl.ds(s * DBIG, DBIG)])
        plsc.subcore_barrier()

        def chunk(j, rows, sem):
            pltpu.make_async_copy(h_hbm.at[sidx.at[j]], rows, sem).wait()
            pltpu.sync_copy(rows, acc.at[didx.at[j]], add=True)
            if with_deg:
                pltpu.sync_copy(ones.at[pl.ds(0, CH)],
                                accd.at[didx.at[j]], add=True)

        for g in range(0, NCH, SCH):
            pltpu.sync_copy(src_hbm.at[w, pl.ds(g, SCH)], sidx)
            pltpu.sync_copy(dst_hbm.at[w, pl.ds(g, SCH)], didx)
            pltpu.async_copy(h_hbm.at[sidx.at[0]], rows0, semA)

            @pl.loop(0, SCH, step=2)
            def _(j):
                pltpu.async_copy(h_hbm.at[sidx.at[j + 1]], rows1, semB)
                chunk(j, rows0, semA)

                @pl.when(j + 2 < SCH)
                def _():
                    pltpu.async_copy(h_hbm.at[sidx.at[j + 2]], rows0, semA)
                chunk(j + 1, rows1, semB)

        plsc.subcore_barrier()

        @pl.when(s < 15)
        def _():
            pltpu.sync_copy(acc.at[pl.ds(s * DBIG, DBIG)],
                            agg_out.at[c, pl.ds(s * DBIG, DBIG)])

        @pl.when(s == 15)
        def _():
            pltpu.sync_copy(acc.at[pl.ds(15 * DBIG, DLAST)],
                            agg_out.at[c, pl.ds(15 * DBIG, DLAST)])
        if with_deg:
            pltpu.sync_copy(accd.at[pl.ds(s * DBIG, DBIG)],
                            deg_out.at[c, 0, pl.ds(s * DBIG, DBIG)])

    fn = pl.kernel(body, out_type=tuple(out_type), mesh=_mesh(),
                   scratch_types=scratch, compiler_params=_SC_PARAMS)
    return fn(h, src3, dst3)


def _seg_sum1(p, src3, dst3):
    scratch = [
        pltpu.VMEM_SHARED((NPAD,), jnp.float32),
        pltpu.VMEM((N,), jnp.float32),
        pltpu.VMEM((SCH1, CH1), jnp.int32),
        pltpu.VMEM((SCH1, CH1), jnp.int32),
        pltpu.VMEM((DBIG,), jnp.float32),
        pltpu.VMEM((CH1,), jnp.float32),
    ]

    def body(p_hbm, src_hbm, dst_hbm, out, accd, table, sidx, didx,
             zdeg, valbuf):
        c = lax.axis_index("c")
        s = lax.axis_index("s")
        w = c * 16 + s

        z16 = jnp.zeros((16,), jnp.float32)

        @pl.loop(0, DBIG // 16)
        def _(i):
            zdeg[pl.ds(i * 16, 16)] = z16
        pltpu.sync_copy(zdeg, accd.at[pl.ds(s * DBIG, DBIG)])
        pltpu.sync_copy(p_hbm, table)
        plsc.subcore_barrier()

        for g in range(0, NCH1, SCH1):
            pltpu.sync_copy(src_hbm.at[w, pl.ds(g, SCH1)], sidx)
            pltpu.sync_copy(dst_hbm.at[w, pl.ds(g, SCH1)], didx)

            @pl.loop(0, SCH1)
            def _(j):
                for q in range(CH1 // 16):
                    iv = sidx[j, pl.ds(q * 16, 16)]
                    valbuf[pl.ds(q * 16, 16)] = plsc.load_gather(table, [iv])
                pltpu.sync_copy(valbuf, accd.at[didx.at[j]], add=True)

        plsc.subcore_barrier()
        pltpu.sync_copy(accd.at[pl.ds(s * DBIG, DBIG)],
                        out.at[c, 0, pl.ds(s * DBIG, DBIG)])

    fn = pl.kernel(body,
                   out_type=jax.ShapeDtypeStruct((2, 1, NPAD), jnp.float32),
                   mesh=_mesh(), scratch_types=scratch,
                   compiler_params=_SC_PARAMS_GATHER)
    return fn(p, src3, dst3)


_RB = 2000


def _tc_layer0_body(aA, aB, dA, dB, h_, wn, ws, bb, o):
    invd = 1.0 / jnp.maximum(dA[...] + dB[...], 1.0)
    agg = (aA[0] + aB[0]) * invd
    out = (jnp.dot(agg, wn[...], preferred_element_type=jnp.float32)
           + jnp.dot(h_[...], ws[...], preferred_element_type=jnp.float32)
           + bb[...])
    o[...] = jnp.maximum(out, 0.0)


def _tc_layer0(aggp, degA, degB, h, Wn, Ws, b2d):
    row = lambda i: (i, 0)
    fixed = lambda i: (0, 0)
    return pl.pallas_call(
        _tc_layer0_body,
        grid=(N // _RB,),
        in_specs=[
            pl.BlockSpec((1, _RB, H), lambda i: (0, i, 0)),
            pl.BlockSpec((1, _RB, H), lambda i: (1, i, 0)),
            pl.BlockSpec((_RB, 1), row),
            pl.BlockSpec((_RB, 1), row),
            pl.BlockSpec((_RB, H), row),
            pl.BlockSpec((H, H), fixed),
            pl.BlockSpec((H, H), fixed),
            pl.BlockSpec((1, H), fixed),
        ],
        out_specs=pl.BlockSpec((_RB, H), row),
        out_shape=jax.ShapeDtypeStruct((N, H), jnp.float32),
    )(aggp, aggp, degA, degB, h, Wn, Ws, b2d)


def _tc_layer1_body(aA, aB, dA, dB, h_, wn, ws, bb, wn2, ws2, b2,
                    oh, op, os):
    invd = 1.0 / jnp.maximum(dA[...] + dB[...], 1.0)
    agg = (aA[0] + aB[0]) * invd
    h2 = (jnp.dot(agg, wn[...], preferred_element_type=jnp.float32)
          + jnp.dot(h_[...], ws[...], preferred_element_type=jnp.float32)
          + bb[...])
    h2 = jnp.maximum(h2, 0.0)
    oh[...] = h2
    op[...] = jnp.sum(h2 * wn2[...], axis=1, keepdims=True)
    os[...] = jnp.sum(h2 * ws2[...], axis=1, keepdims=True) + b2[...]


def _tc_layer1(aggp, degA, degB, h, Wn, Ws, b2d, wn2, ws2, b11):
    row = lambda i: (i, 0)
    fixed = lambda i: (0, 0)
    return pl.pallas_call(
        _tc_layer1_body,
        grid=(N // _RB,),
        in_specs=[
            pl.BlockSpec((1, _RB, H), lambda i: (0, i, 0)),
            pl.BlockSpec((1, _RB, H), lambda i: (1, i, 0)),
            pl.BlockSpec((_RB, 1), row),
            pl.BlockSpec((_RB, 1), row),
            pl.BlockSpec((_RB, H), row),
            pl.BlockSpec((H, H), fixed),
            pl.BlockSpec((H, H), fixed),
            pl.BlockSpec((1, H), fixed),
            pl.BlockSpec((1, H), fixed),
            pl.BlockSpec((1, H), fixed),
            pl.BlockSpec((1, 1), fixed),
        ],
        out_specs=[
            pl.BlockSpec((_RB, H), row),
            pl.BlockSpec((_RB, 1), row),
            pl.BlockSpec((_RB, 1), row),
        ],
        out_shape=[
            jax.ShapeDtypeStruct((N, H), jnp.float32),
            jax.ShapeDtypeStruct((N, 1), jnp.float32),
            jax.ShapeDtypeStruct((N, 1), jnp.float32),
        ],
    )(aggp, aggp, degA, degB, h, Wn, Ws, b2d, wn2, ws2, b11)


def _tc_combine_body(aA, aB, dA, dB, s2, o):
    invd = 1.0 / jnp.maximum(dA[...] + dB[...], 1.0)
    o[...] = (aA[...] + aB[...]) * invd + s2[...]


def _tc_combine(a2A, a2B, degA, degB, s2):
    row = lambda i: (i, 0)
    return pl.pallas_call(
        _tc_combine_body,
        grid=(N // _RB,),
        in_specs=[pl.BlockSpec((_RB, 1), row)] * 5,
        out_specs=pl.BlockSpec((_RB, 1), row),
        out_shape=jax.ShapeDtypeStruct((N, 1), jnp.float32),
    )(a2A, a2B, degA, degB, s2)


def kernel(x, edge_index, Wn0, Ws0, b0, Wn1, Ws1, b1, Wn2, Ws2, b2):
    src = edge_index[0]
    dst = edge_index[1]
    src3 = src.reshape(NW, NCH, CH)
    dst3 = dst.reshape(NW, NCH, CH)
    src3b = src.reshape(NW, NCH1, CH1)
    dst3b = dst.reshape(NW, NCH1, CH1)

    agg0, degp = _seg_sum(x, src3, dst3, with_deg=True)
    degA = degp[0, 0, :N].reshape(N, 1)
    degB = degp[1, 0, :N].reshape(N, 1)

    h1 = _tc_layer0(agg0, degA, degB, x, Wn0, Ws0, b0.reshape(1, H))
    (agg1,) = _seg_sum(h1, src3, dst3, with_deg=False)
    h2, p2, s2 = _tc_layer1(agg1, degA, degB, h1, Wn1, Ws1,
                            b1.reshape(1, H), Wn2.reshape(1, H),
                            Ws2.reshape(1, H), b2.reshape(1, 1))
    agg2p = _seg_sum1(p2.reshape(N), src3b, dst3b)
    a2A = agg2p[0, 0, :N].reshape(N, 1)
    a2B = agg2p[1, 0, :N].reshape(N, 1)
    out = _tc_combine(a2A, a2B, degA, degB, s2)
    return out.reshape(N)
